# Initial kernel scaffold; baseline (speedup 1.0000x reference)
import jax
import jax.numpy as jnp
from jax.experimental import pallas as pl


def kernel(x, pos_emb):
    raise NotImplementedError("write your pallas kernel here")


# speedup vs baseline: 3.3973x; 3.3973x over previous
import jax
import jax.numpy as jnp
from jax.experimental import pallas as pl


_BLK = 1024


def _add_kernel(x_ref, pos_ref, o_ref):
    o_ref[...] = x_ref[...] + pos_ref[...]


def kernel(x, pos_emb):
    batch, length, dim = x.shape
    num_blocks = length // _BLK
    return pl.pallas_call(
        _add_kernel,
        grid=(num_blocks, batch),
        in_specs=[
            pl.BlockSpec((1, _BLK, dim), lambda i, j: (j, i, 0)),
            pl.BlockSpec((_BLK, dim), lambda i, j: (i, 0)),
        ],
        out_specs=pl.BlockSpec((1, _BLK, dim), lambda i, j: (j, i, 0)),
        out_shape=jax.ShapeDtypeStruct(x.shape, x.dtype),
    )(x, pos_emb)


# speedup vs baseline: 3.6286x; 1.0681x over previous
import jax
import jax.numpy as jnp
from jax.experimental import pallas as pl


_BLK = 2048


def _add_kernel(x_ref, pos_ref, o_ref):
    o_ref[...] = x_ref[...] + pos_ref[...]


def kernel(x, pos_emb):
    batch, length, dim = x.shape
    num_blocks = length // _BLK
    return pl.pallas_call(
        _add_kernel,
        grid=(num_blocks, batch),
        in_specs=[
            pl.BlockSpec((1, _BLK, dim), lambda i, j: (j, i, 0)),
            pl.BlockSpec((_BLK, dim), lambda i, j: (i, 0)),
        ],
        out_specs=pl.BlockSpec((1, _BLK, dim), lambda i, j: (j, i, 0)),
        out_shape=jax.ShapeDtypeStruct(x.shape, x.dtype),
    )(x, pos_emb)


# speedup vs baseline: 3.6449x; 1.0045x over previous
import jax
import jax.numpy as jnp
from jax.experimental import pallas as pl


_BLK = 1024


def _add_kernel(x_ref, pos_ref, o_ref):
    o_ref[...] = x_ref[...] + pos_ref[...][None, :, :]


def kernel(x, pos_emb):
    batch, length, dim = x.shape
    num_blocks = length // _BLK
    return pl.pallas_call(
        _add_kernel,
        grid=(num_blocks,),
        in_specs=[
            pl.BlockSpec((batch, _BLK, dim), lambda i: (0, i, 0)),
            pl.BlockSpec((_BLK, dim), lambda i: (i, 0)),
        ],
        out_specs=pl.BlockSpec((batch, _BLK, dim), lambda i: (0, i, 0)),
        out_shape=jax.ShapeDtypeStruct(x.shape, x.dtype),
    )(x, pos_emb)
